# baseline (speedup 1.0000x reference)
import functools

import jax
import jax.numpy as jnp
import numpy as np
from jax import lax
from jax.experimental import pallas as pl
from jax.experimental.pallas import tpu as pltpu
from jax.experimental.pallas import tpu_sc as plsc

B = 2048
N = 8192
NUM_STAGES = 13
D = 768
T = 8
NF = 6
IN_DIM = D // 4 + D // 4 + 4 * NF
F = 2 * D
BT = 1024
NB = B // BT


def _gelu(v):
    return 0.5 * v * (1.0 + lax.erf(v * np.float32(1.0 / np.sqrt(2.0))))


def _sc_gather(stage_tab, stage_idx, pos_tab, pos_idx):
    info = plsc.get_sparse_core_info()
    nw = info.num_cores * info.num_subcores
    b_per_w = B // nw
    dq = 256
    mesh = plsc.VectorSubcoreMesh(core_axis_name="c", subcore_axis_name="s")

    @functools.partial(
        pl.kernel,
        mesh=mesh,
        out_type=(
            jax.ShapeDtypeStruct((B, dq), jnp.float32),
            jax.ShapeDtypeStruct((B, dq), jnp.float32),
        ),
        scratch_types=[
            pltpu.VMEM((b_per_w,), jnp.int32),
            pltpu.VMEM((b_per_w, dq), jnp.float32),
            pltpu.VMEM((b_per_w,), jnp.int32),
            pltpu.VMEM((b_per_w, dq), jnp.float32),
            pltpu.SemaphoreType.DMA,
            pltpu.SemaphoreType.DMA,
        ],
    )
    def k(stab_hbm, sidx_hbm, ptab_hbm, pidx_hbm, se_hbm, pe_hbm,
          sidx_v, srows_v, pidx_v, prows_v, sem_s, sem_p):
        wid = lax.axis_index("s") * info.num_cores + lax.axis_index("c")
        base = wid * b_per_w
        pltpu.sync_copy(sidx_hbm.at[pl.ds(base, b_per_w)], sidx_v)
        pltpu.sync_copy(pidx_hbm.at[pl.ds(base, b_per_w)], pidx_v)
        cp_s = pltpu.async_copy(stab_hbm.at[sidx_v], srows_v, sem_s)
        cp_p = pltpu.async_copy(ptab_hbm.at[pidx_v], prows_v, sem_p)
        cp_s.wait()
        cp_p.wait()
        pltpu.sync_copy(srows_v, se_hbm.at[pl.ds(base, b_per_w)])
        pltpu.sync_copy(prows_v, pe_hbm.at[pl.ds(base, b_per_w)])

    return k(stage_tab, stage_idx, pos_tab, pos_idx)


def _tc_body(x_in_ref, Wp_ref, bp_ref, ln_g_ref, ln_b_ref,
             Wr1s_ref, br1s_ref, Wr2s_ref, br2s_ref,
             Wr1d_ref, br1d_ref, Wr2d_ref, br2d_ref,
             Wt1_ref, bt1_ref, Wt2_ref, bt2_ref,
             Ws_ref, bs_ref, Wd_ref, bd_ref,
             out_s_ref, out_d_ref,
             x_s, tile_s, tile_d, acc_s, acc_d):
    t = pl.program_id(0)
    i = pl.program_id(1)

    @pl.when(t == 0)
    def _stem():
        x_in = x_in_ref[...]
        h = jnp.dot(x_in, Wp_ref[...], preferred_element_type=jnp.float32)
        h = h + bp_ref[...]
        mu = jnp.mean(h, axis=-1, keepdims=True)
        var = jnp.mean((h - mu) ** 2, axis=-1, keepdims=True)
        h = (h - mu) * lax.rsqrt(var + 1e-5) * ln_g_ref[...] + ln_b_ref[...]
        x = _gelu(h)
        x_s[pl.ds(i * BT, BT), :] = x

        iota8 = lax.broadcasted_iota(jnp.int32, (BT, T), 1)

        hs = _gelu(jnp.dot(x, Wr1s_ref[...], preferred_element_type=jnp.float32)
                   + br1s_ref[...])
        ls = jnp.dot(hs, Wr2s_ref[...], preferred_element_type=jnp.float32) \
            + br2s_ref[...]
        ms = jnp.max(ls, axis=-1, keepdims=True)
        ts = jnp.min(jnp.where(ls >= ms, iota8, T), axis=-1, keepdims=True)
        tile_s[pl.ds(i * BT, BT), :] = ts

        hd = _gelu(jnp.dot(x, Wr1d_ref[...], preferred_element_type=jnp.float32)
                   + br1d_ref[...])
        ld = jnp.dot(hd, Wr2d_ref[...], preferred_element_type=jnp.float32) \
            + br2d_ref[...]
        md = jnp.max(ld, axis=-1, keepdims=True)
        td = jnp.min(jnp.where(ld >= md, iota8, T), axis=-1, keepdims=True)
        tile_d[pl.ds(i * BT, BT), :] = td

        acc_s[pl.ds(i * BT, BT), :] = jnp.zeros((BT, 1), jnp.float32)
        acc_d[pl.ds(i * BT, BT), :] = jnp.zeros((BT, 1), jnp.float32)

    x = x_s[pl.ds(i * BT, BT), :]
    v_s = jnp.dot(Wt2_ref[0], Ws_ref[...], preferred_element_type=jnp.float32)
    v_d = jnp.dot(Wt2_ref[0], Wd_ref[...], preferred_element_type=jnp.float32)
    c_s = jnp.sum(bt2_ref[0] * Ws_ref[...].T) + bs_ref[0, 0]
    c_d = jnp.sum(bt2_ref[0] * Wd_ref[...].T) + bd_ref[0, 0]

    th = _gelu(jnp.dot(x.astype(jnp.bfloat16), Wt1_ref[0],
                       preferred_element_type=jnp.float32)
               + bt1_ref[0])
    a_s = jnp.sum(th * v_s.T, axis=-1, keepdims=True) + c_s
    a_d = jnp.sum(th * v_d.T, axis=-1, keepdims=True) + c_d

    sel_s = tile_s[pl.ds(i * BT, BT), :] == t
    sel_d = tile_d[pl.ds(i * BT, BT), :] == t
    new_s = acc_s[pl.ds(i * BT, BT), :] + jnp.where(sel_s, a_s, 0.0)
    new_d = acc_d[pl.ds(i * BT, BT), :] + jnp.where(sel_d, a_d, 0.0)
    acc_s[pl.ds(i * BT, BT), :] = new_s
    acc_d[pl.ds(i * BT, BT), :] = new_d
    out_s_ref[...] = new_s
    out_d_ref[...] = new_d


def _tc_main(x_in, Wp, bp, ln_g, ln_b, Wr1s, br1s, Wr2s, br2s,
             Wr1d, br1d, Wr2d, br2d, Wt1, bt1, Wt2, bt2, Ws, bs, Wd, bd):
    full = lambda shape: pl.BlockSpec(shape, lambda t, i: (0,) * len(shape))
    per_t2 = lambda s2: pl.BlockSpec((1,) + s2[1:], lambda t, i: (t, 0))
    per_t3 = lambda s3: pl.BlockSpec((1,) + s3[1:], lambda t, i: (t, 0, 0))

    grid = (T, NB)
    out_s, out_d = pl.pallas_call(
        _tc_body,
        grid=grid,
        in_specs=[
            pl.BlockSpec((BT, IN_DIM), lambda t, i: (i, 0)),
            full((IN_DIM, D)), full((1, D)), full((1, D)), full((1, D)),
            full((D, D)), full((1, D)), full((D, T)), full((1, T)),
            full((D, D)), full((1, D)), full((D, T)), full((1, T)),
            per_t3(Wt1.shape), per_t3((T, 1, F)),
            per_t3(Wt2.shape), per_t3((T, 1, D)),
            full((D, 1)), full((1, 1)), full((D, 1)), full((1, 1)),
        ],
        out_specs=[
            pl.BlockSpec((BT, 1), lambda t, i: (i, 0)),
            pl.BlockSpec((BT, 1), lambda t, i: (i, 0)),
        ],
        out_shape=[
            jax.ShapeDtypeStruct((B, 1), jnp.float32),
            jax.ShapeDtypeStruct((B, 1), jnp.float32),
        ],
        scratch_shapes=[
            pltpu.VMEM((B, D), jnp.float32),
            pltpu.VMEM((B, 1), jnp.int32),
            pltpu.VMEM((B, 1), jnp.int32),
            pltpu.VMEM((B, 1), jnp.float32),
            pltpu.VMEM((B, 1), jnp.float32),
        ],
    )(x_in, Wp, bp.reshape(1, D), ln_g.reshape(1, D), ln_b.reshape(1, D),
      Wr1s, br1s.reshape(1, D), Wr2s, br2s.reshape(1, T),
      Wr1d, br1d.reshape(1, D), Wr2d, br2d.reshape(1, T),
      Wt1.astype(jnp.bfloat16), bt1.reshape(T, 1, F), Wt2, bt2.reshape(T, 1, D),
      Ws, bs.reshape(1, 1), Wd, bd.reshape(1, 1))
    return out_s[:, 0], out_d[:, 0]


def kernel(stage, pos, a, b, stage_tab, pos_tab, Wp, bp, ln_g, ln_b,
           Wr1s, br1s, Wr2s, br2s, Wr1d, br1d, Wr2d, br2d,
           Wt1, bt1, Wt2, bt2, Ws, bs, Wd, bd):
    pad = 256 - D // 4
    se, pe = _sc_gather(jnp.pad(stage_tab, ((0, 0), (0, pad))),
                        stage.astype(jnp.int32),
                        jnp.pad(pos_tab, ((0, 0), (0, pad))),
                        pos.astype(jnp.int32))
    se = se[:, :D // 4]
    pe = pe[:, :D // 4]
    xn_a = a.astype(jnp.float32)[:, None] * (2.0 * np.pi / 256.0)
    xn_b = b.astype(jnp.float32)[:, None] * (2.0 * np.pi / 256.0)
    freqs = (2.0 ** jnp.arange(NF, dtype=jnp.float32))[None, :]
    af = jnp.concatenate([jnp.sin(xn_a * freqs), jnp.cos(xn_a * freqs)], -1)
    bf = jnp.concatenate([jnp.sin(xn_b * freqs), jnp.cos(xn_b * freqs)], -1)
    x_in = jnp.concatenate([se, pe, af, bf], axis=-1)
    return _tc_main(x_in, Wp, bp, ln_g, ln_b, Wr1s, br1s, Wr2s, br2s,
                    Wr1d, br1d, Wr2d, br2d, Wt1, bt1, Wt2, bt2, Ws, bs, Wd, bd)


# speedup vs baseline: 1.0201x; 1.0201x over previous
import functools

import jax
import jax.numpy as jnp
import numpy as np
from jax import lax
from jax.experimental import pallas as pl
from jax.experimental.pallas import tpu as pltpu
from jax.experimental.pallas import tpu_sc as plsc

B = 2048
N = 8192
NUM_STAGES = 13
D = 768
T = 8
NF = 6
IN_DIM = D // 4 + D // 4 + 4 * NF
F = 2 * D
BT = 1024
NB = B // BT
BT3 = 256
R = 4096 + T * BT3
NBLK = R // BT3
NCH = 16
CHS = B // NCH
NW = 32


def _gelu(v):
    return 0.5 * v * (1.0 + lax.erf(v * np.float32(1.0 / np.sqrt(2.0))))


def _sc_gather(stage_tab, stage_idx, pos_tab, pos_idx):
    info = plsc.get_sparse_core_info()
    nw = info.num_cores * info.num_subcores
    b_per_w = B // nw
    dq = 256
    mesh = plsc.VectorSubcoreMesh(core_axis_name="c", subcore_axis_name="s")

    @functools.partial(
        pl.kernel,
        mesh=mesh,
        out_type=(
            jax.ShapeDtypeStruct((B, dq), jnp.float32),
            jax.ShapeDtypeStruct((B, dq), jnp.float32),
        ),
        scratch_types=[
            pltpu.VMEM((b_per_w,), jnp.int32),
            pltpu.VMEM((b_per_w, dq), jnp.float32),
            pltpu.VMEM((b_per_w,), jnp.int32),
            pltpu.VMEM((b_per_w, dq), jnp.float32),
            pltpu.SemaphoreType.DMA,
            pltpu.SemaphoreType.DMA,
        ],
    )
    def k(stab_hbm, sidx_hbm, ptab_hbm, pidx_hbm, se_hbm, pe_hbm,
          sidx_v, srows_v, pidx_v, prows_v, sem_s, sem_p):
        wid = lax.axis_index("s") * info.num_cores + lax.axis_index("c")
        base = wid * b_per_w
        pltpu.sync_copy(sidx_hbm.at[pl.ds(base, b_per_w)], sidx_v)
        pltpu.sync_copy(pidx_hbm.at[pl.ds(base, b_per_w)], pidx_v)
        cp_s = pltpu.async_copy(stab_hbm.at[sidx_v], srows_v, sem_s)
        cp_p = pltpu.async_copy(ptab_hbm.at[pidx_v], prows_v, sem_p)
        cp_s.wait()
        cp_p.wait()
        pltpu.sync_copy(srows_v, se_hbm.at[pl.ds(base, b_per_w)])
        pltpu.sync_copy(prows_v, pe_hbm.at[pl.ds(base, b_per_w)])

    return k(stage_tab, stage_idx, pos_tab, pos_idx)


def _stem_body(x_in_ref, Wp_ref, bp_ref, ln_g_ref, ln_b_ref,
               Wr1s_ref, br1s_ref, Wr2s_ref, br2s_ref,
               Wr1d_ref, br1d_ref, Wr2d_ref, br2d_ref,
               x_ref, ts_ref, td_ref):
    x_in = x_in_ref[...]
    h = jnp.dot(x_in, Wp_ref[...], preferred_element_type=jnp.float32)
    h = h + bp_ref[...]
    mu = jnp.mean(h, axis=-1, keepdims=True)
    var = jnp.mean((h - mu) ** 2, axis=-1, keepdims=True)
    h = (h - mu) * lax.rsqrt(var + 1e-5) * ln_g_ref[...] + ln_b_ref[...]
    x = _gelu(h)
    x_ref[...] = x

    iota8 = lax.broadcasted_iota(jnp.int32, (BT, T), 1)

    hs = _gelu(jnp.dot(x, Wr1s_ref[...], preferred_element_type=jnp.float32)
               + br1s_ref[...])
    ls = jnp.dot(hs, Wr2s_ref[...], preferred_element_type=jnp.float32) \
        + br2s_ref[...]
    ms = jnp.max(ls, axis=-1, keepdims=True)
    ts_ref[...] = jnp.min(jnp.where(ls >= ms, iota8, T), axis=-1, keepdims=True)

    hd = _gelu(jnp.dot(x, Wr1d_ref[...], preferred_element_type=jnp.float32)
               + br1d_ref[...])
    ld = jnp.dot(hd, Wr2d_ref[...], preferred_element_type=jnp.float32) \
        + br2d_ref[...]
    md = jnp.max(ld, axis=-1, keepdims=True)
    td_ref[...] = jnp.min(jnp.where(ld >= md, iota8, T), axis=-1, keepdims=True)


def _tc_stem(x_in, Wp, bp, ln_g, ln_b, Wr1s, br1s, Wr2s, br2s,
             Wr1d, br1d, Wr2d, br2d):
    full = lambda shape: pl.BlockSpec(shape, lambda i: (0,) * len(shape))
    return pl.pallas_call(
        _stem_body,
        grid=(NB,),
        in_specs=[
            pl.BlockSpec((BT, IN_DIM), lambda i: (i, 0)),
            full((IN_DIM, D)), full((1, D)), full((1, D)), full((1, D)),
            full((D, D)), full((1, D)), full((D, T)), full((1, T)),
            full((D, D)), full((1, D)), full((D, T)), full((1, T)),
        ],
        out_specs=[
            pl.BlockSpec((BT, D), lambda i: (i, 0)),
            pl.BlockSpec((BT, 1), lambda i: (i, 0)),
            pl.BlockSpec((BT, 1), lambda i: (i, 0)),
        ],
        out_shape=[
            jax.ShapeDtypeStruct((B, D), jnp.float32),
            jax.ShapeDtypeStruct((B, 1), jnp.int32),
            jax.ShapeDtypeStruct((B, 1), jnp.int32),
        ],
    )(x_in, Wp, bp.reshape(1, D), ln_g.reshape(1, D), ln_b.reshape(1, D),
      Wr1s, br1s.reshape(1, D), Wr2s, br2s.reshape(1, T),
      Wr1d, br1d.reshape(1, D), Wr2d, br2d.reshape(1, T))


def _plan_body(ts_ref, td_ref, pos_ref, be_ref, sb_ref):
    S = 2 * B
    keys = jnp.concatenate([ts_ref[...], td_ref[...]], axis=0)
    iota8s = lax.broadcasted_iota(jnp.int32, (S, T), 1)
    oh = jnp.where(keys == iota8s, 1.0, 0.0).astype(jnp.float32)

    cm = jnp.where(
        lax.broadcasted_iota(jnp.int32, (NW, S), 1) // CHS
        == lax.broadcasted_iota(jnp.int32, (NW, S), 0),
        1.0, 0.0).astype(jnp.float32)
    C = jnp.dot(cm, oh, preferred_element_type=jnp.float32)

    tri_nw = jnp.where(
        lax.broadcasted_iota(jnp.int32, (NW, NW), 1)
        < lax.broadcasted_iota(jnp.int32, (NW, NW), 0),
        1.0, 0.0).astype(jnp.float32)
    prefix = jnp.dot(tri_nw, C, preferred_element_type=jnp.float32)

    total = jnp.sum(C, axis=0, keepdims=True)
    half = jnp.where(
        lax.broadcasted_iota(jnp.int32, (1, NW), 1) < NCH, 1.0, 0.0)
    tot_s = jnp.dot(half.astype(jnp.float32), C,
                    preferred_element_type=jnp.float32)

    tot_i = total.astype(jnp.int32)
    padded = ((tot_i + (BT3 - 1)) // BT3) * BT3
    triu8 = jnp.where(
        lax.broadcasted_iota(jnp.int32, (T, T), 0)
        < lax.broadcasted_iota(jnp.int32, (T, T), 1),
        1.0, 0.0).astype(jnp.float32)
    start = jnp.dot(padded.astype(jnp.float32), triu8,
                    preferred_element_type=jnp.float32)
    sb = start + tot_s

    base = start + prefix

    tri_ch = jnp.where(
        lax.broadcasted_iota(jnp.int32, (CHS, CHS), 1)
        < lax.broadcasted_iota(jnp.int32, (CHS, CHS), 0),
        1.0, 0.0).astype(jnp.float32)
    pos_e = jnp.concatenate(
        [jnp.dot(tri_ch, oh[c * CHS:(c + 1) * CHS],
                 preferred_element_type=jnp.float32)
         + base[c:c + 1, :]
         for c in range(NW)], axis=0)

    posf = jnp.sum(oh * pos_e, axis=-1, keepdims=True)
    pos_ref[...] = posf.astype(jnp.int32)

    sblk = start.astype(jnp.int32) // BT3
    jb = lax.broadcasted_iota(jnp.int32, (NW, T), 0)
    cmp = jnp.where(jb >= sblk, 1, 0)
    be = jnp.sum(cmp, axis=-1, keepdims=True) - 1
    be = jnp.clip(be, 0, T - 1)
    iota8b = lax.broadcasted_iota(jnp.int32, (NW, T), 1)
    oh_be = jnp.where(be == iota8b, 1.0, 0.0).astype(jnp.float32)
    sbj = jnp.sum(oh_be * sb, axis=-1, keepdims=True)
    be_ref[...] = be
    sb_ref[...] = sbj.astype(jnp.int32)


def _tc_plan(ts, td):
    full2 = lambda shape: pl.BlockSpec(shape, lambda: (0, 0))
    return pl.pallas_call(
        _plan_body,
        in_specs=[full2((B, 1)), full2((B, 1))],
        out_specs=[full2((2 * B, 1)), full2((NW, 1)), full2((NW, 1))],
        out_shape=[
            jax.ShapeDtypeStruct((2 * B, 1), jnp.int32),
            jax.ShapeDtypeStruct((NW, 1), jnp.int32),
            jax.ShapeDtypeStruct((NW, 1), jnp.int32),
        ],
    )(ts, td)


def _sc_scatter_rows(posmap, x):
    info = plsc.get_sparse_core_info()
    mesh = plsc.VectorSubcoreMesh(core_axis_name="c", subcore_axis_name="s")

    @functools.partial(
        pl.kernel,
        mesh=mesh,
        out_type=jax.ShapeDtypeStruct((R, D), jnp.float32),
        scratch_types=[
            pltpu.VMEM((CHS,), jnp.int32),
            pltpu.VMEM((CHS, D), jnp.float32),
            pltpu.SemaphoreType.DMA,
        ],
    )
    def k(posmap_hbm, x_hbm, xs_hbm, pm_v, xrows_v, sem):
        wid = lax.axis_index("s") * info.num_cores + lax.axis_index("c")
        chunk = wid % NCH
        pltpu.sync_copy(posmap_hbm.at[pl.ds(wid * CHS, CHS)], pm_v)
        pltpu.sync_copy(x_hbm.at[pl.ds(chunk * CHS, CHS)], xrows_v)
        pltpu.async_copy(xrows_v, xs_hbm.at[pm_v], sem).wait()

    return k(posmap, x)


def _expert_body(be_ref, sb_ref, xs_ref, Wt1_ref, bt1_ref, Wt2_ref, bt2_ref,
                 Wsd_ref, bsd_ref, pred_ref):
    j = pl.program_id(0)
    v_sd = jnp.dot(Wt2_ref[0], Wsd_ref[...], preferred_element_type=jnp.float32)
    c_sd = jnp.dot(bt2_ref[0], Wsd_ref[...], preferred_element_type=jnp.float32) \
        + bsd_ref[...]
    th = _gelu(jnp.dot(xs_ref[...].astype(jnp.bfloat16), Wt1_ref[0],
                       preferred_element_type=jnp.float32) + bt1_ref[0])
    a = jnp.dot(th, v_sd, preferred_element_type=jnp.float32) + c_sd
    rowpos = j * BT3 + lax.broadcasted_iota(jnp.int32, (BT3, 1), 0)
    sel = jnp.where(rowpos >= sb_ref[j], a[:, 1:2], a[:, 0:1])
    pred_ref[...] = jnp.broadcast_to(sel, (BT3, 128))


def _tc_experts(block_expert, sb_blk, x_sorted, Wt1, bt1, Wt2, bt2,
                Ws, bs, Wd, bd):
    full = lambda shape: pl.BlockSpec(shape, lambda j, be, sb: (0,) * len(shape))
    grid_spec = pltpu.PrefetchScalarGridSpec(
        num_scalar_prefetch=2,
        grid=(NBLK,),
        in_specs=[
            pl.BlockSpec((BT3, D), lambda j, be, sb: (j, 0)),
            pl.BlockSpec((1, D, F), lambda j, be, sb: (be[j], 0, 0)),
            pl.BlockSpec((1, 1, F), lambda j, be, sb: (be[j], 0, 0)),
            pl.BlockSpec((1, F, D), lambda j, be, sb: (be[j], 0, 0)),
            pl.BlockSpec((1, 1, D), lambda j, be, sb: (be[j], 0, 0)),
            full((D, 2)), full((1, 2)),
        ],
        out_specs=pl.BlockSpec((BT3, 128), lambda j, be, sb: (j, 0)),
    )
    Wsd = jnp.concatenate([Ws, Wd], axis=1)
    bsd = jnp.stack([bs[0], bd[0]]).reshape(1, 2)
    return pl.pallas_call(
        _expert_body,
        grid_spec=grid_spec,
        out_shape=jax.ShapeDtypeStruct((R, 128), jnp.float32),
    )(block_expert, sb_blk, x_sorted,
      Wt1.astype(jnp.bfloat16), bt1.reshape(T, 1, F), Wt2,
      bt2.reshape(T, 1, D), Wsd, bsd)


def _sc_unpermute(pred2d, posmap):
    info = plsc.get_sparse_core_info()
    mesh = plsc.VectorSubcoreMesh(core_axis_name="c", subcore_axis_name="s")

    @functools.partial(
        pl.kernel,
        mesh=mesh,
        out_type=(
            jax.ShapeDtypeStruct((B, 128), jnp.float32),
            jax.ShapeDtypeStruct((B, 128), jnp.float32),
        ),
        scratch_types=[
            pltpu.VMEM((CHS,), jnp.int32),
            pltpu.VMEM((CHS, 128), jnp.float32),
            pltpu.SemaphoreType.DMA,
        ],
    )
    def k(pred_hbm, posmap_hbm, sum_hbm, diff_hbm, pm_v, rows_v, sem):
        wid = lax.axis_index("s") * info.num_cores + lax.axis_index("c")
        route = wid // NCH
        chunk = wid % NCH
        pltpu.sync_copy(posmap_hbm.at[pl.ds(wid * CHS, CHS)], pm_v)
        pltpu.async_copy(pred_hbm.at[pm_v], rows_v, sem).wait()

        @pl.when(route == 0)
        def _s():
            pltpu.sync_copy(rows_v, sum_hbm.at[pl.ds(chunk * CHS, CHS)])

        @pl.when(route == 1)
        def _d():
            pltpu.sync_copy(rows_v, diff_hbm.at[pl.ds(chunk * CHS, CHS)])

    return k(pred2d, posmap)


def kernel(stage, pos, a, b, stage_tab, pos_tab, Wp, bp, ln_g, ln_b,
           Wr1s, br1s, Wr2s, br2s, Wr1d, br1d, Wr2d, br2d,
           Wt1, bt1, Wt2, bt2, Ws, bs, Wd, bd):
    pad = 256 - D // 4
    se, pe = _sc_gather(jnp.pad(stage_tab, ((0, 0), (0, pad))),
                        stage.astype(jnp.int32),
                        jnp.pad(pos_tab, ((0, 0), (0, pad))),
                        pos.astype(jnp.int32))
    se = se[:, :D // 4]
    pe = pe[:, :D // 4]
    xn_a = a.astype(jnp.float32)[:, None] * (2.0 * np.pi / 256.0)
    xn_b = b.astype(jnp.float32)[:, None] * (2.0 * np.pi / 256.0)
    freqs = (2.0 ** jnp.arange(NF, dtype=jnp.float32))[None, :]
    af = jnp.concatenate([jnp.sin(xn_a * freqs), jnp.cos(xn_a * freqs)], -1)
    bf = jnp.concatenate([jnp.sin(xn_b * freqs), jnp.cos(xn_b * freqs)], -1)
    x_in = jnp.concatenate([se, pe, af, bf], axis=-1)

    x, ts, td = _tc_stem(x_in, Wp, bp, ln_g, ln_b,
                         Wr1s, br1s, Wr2s, br2s, Wr1d, br1d, Wr2d, br2d)
    posmap2d, be2d, sb2d = _tc_plan(ts, td)
    posmap = posmap2d[:, 0]
    xs = _sc_scatter_rows(posmap, x)
    pred2d = _tc_experts(be2d[:, 0], sb2d[:, 0], xs,
                         Wt1, bt1, Wt2, bt2, Ws, bs, Wd, bd)
    sum2d, diff2d = _sc_unpermute(pred2d, posmap)
    return sum2d[:, 0], diff2d[:, 0]
